# baseline (speedup 1.0000x reference)
import jax
import jax.numpy as jnp
import numpy as np
from jax.experimental import pallas as pl
from jax.experimental.pallas import tpu as pltpu

MAX_SCALE = 4
NUM_ANCHORS = 8
EPS = 1e-6
HBLK = 56
HALO = 4


def _shift_w(a, d):
    if d > 0:
        return jnp.concatenate([jnp.zeros_like(a[:, :d]), a[:, :-d]], axis=1)
    if d < 0:
        return jnp.concatenate([a[:, -d:], jnp.zeros_like(a[:, :-d])], axis=1)
    return a


def _fused_kernel(c_ref, anch_ref, wid_ref, xe_ref, o_ref):
    hi = pl.program_id(1)
    nblk = pl.num_programs(1)
    xe = xe_ref[0, 0]
    mu = jnp.maximum(xe, 0.0) + EPS
    row = jax.lax.broadcasted_iota(jnp.int32, mu.shape, 0)
    oob = ((hi == 0) & (row < HALO)) | ((hi == nblk - 1) & (row >= HALO + HBLK))
    mu = jnp.where(oob, 0.0, mu)

    alpha = jnp.zeros((HBLK,) + mu.shape[1:], mu.dtype)
    k = mu + _shift_w(mu, 1) + _shift_w(mu, -1)
    for r in range(1, MAX_SCALE + 1):
        if r > 1:
            k = k + _shift_w(mu, r) + _shift_w(mu, -r)
        lo = HALO - r
        if r == 1:
            p2 = k[lo:lo + HBLK + 1] + k[lo + 1:lo + HBLK + 2]
            box = p2[:HBLK] + k[lo + 2:lo + 2 + HBLK]
        else:
            psz = k.shape[0] - lo - 1
            p2 = k[lo:lo + psz] + k[lo + 1:lo + 1 + psz]
            p4 = p2[:psz - 2] + p2[2:psz]
            if r == 2:
                box = p4[:HBLK] + k[lo + 4:lo + 4 + HBLK]
            elif r == 3:
                box = p4[:HBLK] + p2[4:4 + HBLK] + k[lo + 6:lo + 6 + HBLK]
            else:
                box = p4[:HBLK] + p4[4:4 + HBLK] + k[lo + 8:lo + 8 + HBLK]
        alpha = alpha + c_ref[r - 1] * jnp.log2(box + EPS)

    acc = jnp.zeros_like(alpha)
    for a in range(NUM_ANCHORS):
        t = (alpha - anch_ref[a]) * wid_ref[a]
        acc = acc + jnp.minimum(jnp.abs(t), 1.0)
    o_ref[0] = (xe[HALO:HALO + HBLK] + float(NUM_ANCHORS)) - acc


@jax.jit
def kernel(x, scale_w, anchors, widths):
    b, h, w, c = x.shape
    nblk = h // HBLK

    sw = jax.nn.softmax(scale_w)
    log_r = jnp.log(jnp.asarray([2.0 * r + 1.0 for r in range(1, MAX_SCALE + 1)],
                                dtype=x.dtype))
    dev = log_r - jnp.sum(sw * log_r)
    var = jnp.sum(sw * dev * dev)
    coef = (sw * dev) * (float(np.log(2.0)) / (var + EPS))

    xp = jnp.pad(x, ((0, 0), (HALO, HALO), (0, 0), (0, 0)))
    xe = jnp.stack([xp[:, i * HBLK:i * HBLK + HBLK + 2 * HALO]
                    for i in range(nblk)], axis=1)

    anch = jnp.transpose(anchors).reshape(NUM_ANCHORS, 1, c)
    wid = jnp.transpose(widths).reshape(NUM_ANCHORS, 1, c)

    return pl.pallas_call(
        _fused_kernel,
        grid=(b, nblk),
        in_specs=[
            pl.BlockSpec(memory_space=pltpu.SMEM),
            pl.BlockSpec((NUM_ANCHORS, 1, c), lambda bi, hi: (0, 0, 0)),
            pl.BlockSpec((NUM_ANCHORS, 1, c), lambda bi, hi: (0, 0, 0)),
            pl.BlockSpec((1, 1, HBLK + 2 * HALO, w, c),
                         lambda bi, hi: (bi, hi, 0, 0, 0)),
        ],
        out_specs=pl.BlockSpec((1, HBLK, w, c), lambda bi, hi: (bi, hi, 0, 0)),
        out_shape=jax.ShapeDtypeStruct(x.shape, x.dtype),
    )(coef, anch, wid, xe)


# speedup vs baseline: 1.6852x; 1.6852x over previous
import jax
import jax.numpy as jnp
import numpy as np
from jax.experimental import pallas as pl
from jax.experimental.pallas import tpu as pltpu

MAX_SCALE = 4
NUM_ANCHORS = 8
EPS = 1e-6
HBLK = 56
HALO = 4
SLAB = HBLK + 2 * HALO


def _shift_w(a, d):
    if d > 0:
        return jnp.concatenate([jnp.zeros_like(a[:, :d]), a[:, :-d]], axis=1)
    if d < 0:
        return jnp.concatenate([a[:, -d:], jnp.zeros_like(a[:, :-d])], axis=1)
    return a


def _start_slab_copy(x_ref, xe_scr, sem, bi, hi, nblk, slot):
    h = x_ref.shape[1]

    @pl.when(hi == 0)
    def _():
        pltpu.make_async_copy(
            x_ref.at[bi, pl.ds(0, SLAB - HALO)],
            xe_scr.at[slot, pl.ds(HALO, SLAB - HALO)],
            sem.at[slot],
        ).start()

    @pl.when(hi == nblk - 1)
    def _():
        pltpu.make_async_copy(
            x_ref.at[bi, pl.ds(h - (SLAB - HALO), SLAB - HALO)],
            xe_scr.at[slot, pl.ds(0, SLAB - HALO)],
            sem.at[slot],
        ).start()

    @pl.when((hi > 0) & (hi < nblk - 1))
    def _():
        pltpu.make_async_copy(
            x_ref.at[bi, pl.ds(hi * HBLK - HALO, SLAB)],
            xe_scr.at[slot, pl.ds(0, SLAB)],
            sem.at[slot],
        ).start()


def _wait_slab_copy(x_ref, xe_scr, sem, hi, nblk, slot):
    h = x_ref.shape[1]

    @pl.when((hi == 0) | (hi == nblk - 1))
    def _():
        pltpu.make_async_copy(
            x_ref.at[0, pl.ds(0, SLAB - HALO)],
            xe_scr.at[slot, pl.ds(0, SLAB - HALO)],
            sem.at[slot],
        ).wait()

    @pl.when((hi > 0) & (hi < nblk - 1))
    def _():
        pltpu.make_async_copy(
            x_ref.at[0, pl.ds(0, SLAB)],
            xe_scr.at[slot, pl.ds(0, SLAB)],
            sem.at[slot],
        ).wait()


def _fused_kernel(c_ref, anch_ref, wid_ref, x_ref, o_ref, xe_scr, sem):
    bi = pl.program_id(0)
    hi = pl.program_id(1)
    nb = pl.num_programs(0)
    nblk = pl.num_programs(1)
    step = bi * nblk + hi
    slot = jax.lax.rem(step, 2)

    @pl.when(step == 0)
    def _():
        _start_slab_copy(x_ref, xe_scr, sem, bi, hi, nblk, slot)

    nxt = step + 1
    nxt_bi = nxt // nblk
    nxt_hi = jax.lax.rem(nxt, nblk)

    @pl.when(nxt < nb * nblk)
    def _():
        _start_slab_copy(x_ref, xe_scr, sem, nxt_bi, nxt_hi, nblk, 1 - slot)

    _wait_slab_copy(x_ref, xe_scr, sem, hi, nblk, slot)

    xe = xe_scr[slot]
    mu = jnp.maximum(xe, 0.0) + EPS
    row = jax.lax.broadcasted_iota(jnp.int32, mu.shape, 0)
    oob = ((hi == 0) & (row < HALO)) | ((hi == nblk - 1) & (row >= HALO + HBLK))
    mu = jnp.where(oob, 0.0, mu)

    alpha = jnp.zeros((HBLK,) + mu.shape[1:], mu.dtype)
    k = mu + _shift_w(mu, 1) + _shift_w(mu, -1)
    for r in range(1, MAX_SCALE + 1):
        if r > 1:
            k = k + _shift_w(mu, r) + _shift_w(mu, -r)
        lo = HALO - r
        if r == 1:
            p2 = k[lo:lo + HBLK + 1] + k[lo + 1:lo + HBLK + 2]
            box = p2[:HBLK] + k[lo + 2:lo + 2 + HBLK]
        else:
            psz = k.shape[0] - lo - 1
            p2 = k[lo:lo + psz] + k[lo + 1:lo + 1 + psz]
            p4 = p2[:psz - 2] + p2[2:psz]
            if r == 2:
                box = p4[:HBLK] + k[lo + 4:lo + 4 + HBLK]
            elif r == 3:
                box = p4[:HBLK] + p2[4:4 + HBLK] + k[lo + 6:lo + 6 + HBLK]
            else:
                box = p4[:HBLK] + p4[4:4 + HBLK] + k[lo + 8:lo + 8 + HBLK]
        alpha = alpha + c_ref[r - 1] * jnp.log2(box + EPS)

    acc = jnp.zeros_like(alpha)
    for a in range(NUM_ANCHORS):
        t = (alpha - anch_ref[a]) * wid_ref[a]
        acc = acc + jnp.minimum(jnp.abs(t), 1.0)
    o_ref[0] = (xe[HALO:HALO + HBLK] + float(NUM_ANCHORS)) - acc


@jax.jit
def kernel(x, scale_w, anchors, widths):
    b, h, w, c = x.shape
    nblk = h // HBLK

    sw = jax.nn.softmax(scale_w)
    log_r = jnp.log(jnp.asarray([2.0 * r + 1.0 for r in range(1, MAX_SCALE + 1)],
                                dtype=x.dtype))
    dev = log_r - jnp.sum(sw * log_r)
    var = jnp.sum(sw * dev * dev)
    coef = (sw * dev) * (float(np.log(2.0)) / (var + EPS))

    anch = jnp.transpose(anchors).reshape(NUM_ANCHORS, 1, c)
    wid = jnp.transpose(widths).reshape(NUM_ANCHORS, 1, c)

    return pl.pallas_call(
        _fused_kernel,
        grid=(b, nblk),
        in_specs=[
            pl.BlockSpec(memory_space=pltpu.SMEM),
            pl.BlockSpec((NUM_ANCHORS, 1, c), lambda bi, hi: (0, 0, 0)),
            pl.BlockSpec((NUM_ANCHORS, 1, c), lambda bi, hi: (0, 0, 0)),
            pl.BlockSpec(memory_space=pl.ANY),
        ],
        out_specs=pl.BlockSpec((1, HBLK, w, c), lambda bi, hi: (bi, hi, 0, 0)),
        out_shape=jax.ShapeDtypeStruct(x.shape, x.dtype),
        scratch_shapes=[
            pltpu.VMEM((2, SLAB, w, c), x.dtype),
            pltpu.SemaphoreType.DMA((2,)),
        ],
    )(coef, anch, wid, x)


# speedup vs baseline: 1.8676x; 1.1082x over previous
import jax
import jax.numpy as jnp
import numpy as np
from jax.experimental import pallas as pl
from jax.experimental.pallas import tpu as pltpu

MAX_SCALE = 4
NUM_ANCHORS = 8
EPS = 1e-6
HBLK = 56
HALO = 4
SLAB = HBLK + 2 * HALO


def _shift_w(a, d):
    if d > 0:
        return jnp.concatenate([jnp.zeros_like(a[:, :, :d]), a[:, :, :-d]], axis=2)
    if d < 0:
        return jnp.concatenate([a[:, :, -d:], jnp.zeros_like(a[:, :, :-d])], axis=2)
    return a


def _start_slab_copy(x_ref, xe_scr, sem, bi, hi, nblk, slot):
    h = x_ref.shape[1]

    @pl.when(hi == 0)
    def _():
        pltpu.make_async_copy(
            x_ref.at[bi, pl.ds(0, SLAB - HALO)],
            xe_scr.at[slot, pl.ds(HALO, SLAB - HALO)],
            sem.at[slot],
        ).start()

    @pl.when(hi == nblk - 1)
    def _():
        pltpu.make_async_copy(
            x_ref.at[bi, pl.ds(h - (SLAB - HALO), SLAB - HALO)],
            xe_scr.at[slot, pl.ds(0, SLAB - HALO)],
            sem.at[slot],
        ).start()

    @pl.when((hi > 0) & (hi < nblk - 1))
    def _():
        pltpu.make_async_copy(
            x_ref.at[bi, pl.ds(hi * HBLK - HALO, SLAB)],
            xe_scr.at[slot, pl.ds(0, SLAB)],
            sem.at[slot],
        ).start()


def _wait_slab_copy(x_ref, xe_scr, sem, hi, nblk, slot):
    @pl.when((hi == 0) | (hi == nblk - 1))
    def _():
        pltpu.make_async_copy(
            x_ref.at[0, pl.ds(0, SLAB - HALO)],
            xe_scr.at[slot, pl.ds(0, SLAB - HALO)],
            sem.at[slot],
        ).wait()

    @pl.when((hi > 0) & (hi < nblk - 1))
    def _():
        pltpu.make_async_copy(
            x_ref.at[0, pl.ds(0, SLAB)],
            xe_scr.at[slot, pl.ds(0, SLAB)],
            sem.at[slot],
        ).wait()


def _fused_kernel(c_ref, anch_ref, wid_ref, x_ref, o_ref, xe_scr, sem):
    bi = pl.program_id(0)
    hi = pl.program_id(1)
    nb = pl.num_programs(0)
    nblk = pl.num_programs(1)
    step = bi * nblk + hi
    slot = jax.lax.rem(step, 2)

    @pl.when(step == 0)
    def _():
        _start_slab_copy(x_ref, xe_scr, sem, bi, hi, nblk, slot)

    nxt = step + 1
    nxt_bi = nxt // nblk
    nxt_hi = jax.lax.rem(nxt, nblk)

    @pl.when(nxt < nb * nblk)
    def _():
        _start_slab_copy(x_ref, xe_scr, sem, nxt_bi, nxt_hi, nblk, 1 - slot)

    _wait_slab_copy(x_ref, xe_scr, sem, hi, nblk, slot)

    xe = xe_scr[slot]
    mu = jnp.maximum(xe, 0.0) + EPS
    row = jax.lax.broadcasted_iota(jnp.int32, mu.shape, 0)
    oob = ((hi == 0) & (row < HALO)) | ((hi == nblk - 1) & (row >= HALO + HBLK))
    mu = jnp.where(oob, 0.0, mu)

    alpha = jnp.zeros((HBLK,) + mu.shape[1:], mu.dtype)
    k = mu + _shift_w(mu, 1) + _shift_w(mu, -1)
    for r in range(1, MAX_SCALE + 1):
        if r > 1:
            k = k + _shift_w(mu, r) + _shift_w(mu, -r)
        lo = HALO - r
        if r == 1:
            p2 = k[lo:lo + HBLK + 1] + k[lo + 1:lo + HBLK + 2]
            box = p2[:HBLK] + k[lo + 2:lo + 2 + HBLK]
        else:
            psz = k.shape[0] - lo - 1
            p2 = k[lo:lo + psz] + k[lo + 1:lo + 1 + psz]
            p4 = p2[:psz - 2] + p2[2:psz]
            if r == 2:
                box = p4[:HBLK] + k[lo + 4:lo + 4 + HBLK]
            elif r == 3:
                box = p4[:HBLK] + p2[4:4 + HBLK] + k[lo + 6:lo + 6 + HBLK]
            else:
                box = p4[:HBLK] + p4[4:4 + HBLK] + k[lo + 8:lo + 8 + HBLK]
        alpha = alpha + c_ref[r - 1] * jnp.log2(box + EPS)

    acc = jnp.zeros_like(alpha)
    for a in range(NUM_ANCHORS):
        t = (alpha - anch_ref[a]) * wid_ref[a]
        acc = acc + jnp.minimum(jnp.abs(t), 1.0)
    o_ref[0] = (xe[HALO:HALO + HBLK] + float(NUM_ANCHORS)) - acc


@jax.jit
def kernel(x, scale_w, anchors, widths):
    b, h, w, c = x.shape
    nblk = h // HBLK

    sw = jax.nn.softmax(scale_w)
    log_r = jnp.log(jnp.asarray([2.0 * r + 1.0 for r in range(1, MAX_SCALE + 1)],
                                dtype=x.dtype))
    dev = log_r - jnp.sum(sw * log_r)
    var = jnp.sum(sw * dev * dev)
    coef = (sw * dev) * (float(np.log(2.0)) / (var + EPS))

    anch = jnp.transpose(anchors).reshape(NUM_ANCHORS, c, 1)
    wid = jnp.transpose(widths).reshape(NUM_ANCHORS, c, 1)

    xt = jnp.transpose(x, (0, 1, 3, 2))

    out_t = pl.pallas_call(
        _fused_kernel,
        grid=(b, nblk),
        in_specs=[
            pl.BlockSpec(memory_space=pltpu.SMEM),
            pl.BlockSpec((NUM_ANCHORS, c, 1), lambda bi, hi: (0, 0, 0)),
            pl.BlockSpec((NUM_ANCHORS, c, 1), lambda bi, hi: (0, 0, 0)),
            pl.BlockSpec(memory_space=pl.ANY),
        ],
        out_specs=pl.BlockSpec((1, HBLK, c, w), lambda bi, hi: (bi, hi, 0, 0)),
        out_shape=jax.ShapeDtypeStruct((b, h, c, w), x.dtype),
        scratch_shapes=[
            pltpu.VMEM((2, SLAB, c, w), x.dtype),
            pltpu.SemaphoreType.DMA((2,)),
        ],
    )(coef, anch, wid, xt)
    return jnp.transpose(out_t, (0, 1, 3, 2))


# speedup vs baseline: 2.4133x; 1.2922x over previous
import jax
import jax.numpy as jnp
import numpy as np
from jax.experimental import pallas as pl
from jax.experimental.pallas import tpu as pltpu

MAX_SCALE = 4
NUM_ANCHORS = 8
EPS = 1e-6
HBLK = 56
HALO = 4
SLAB = HBLK + 2 * HALO


def _shift_w(a, d):
    if d > 0:
        return jnp.concatenate([jnp.zeros_like(a[:, :d]), a[:, :-d]], axis=1)
    if d < 0:
        return jnp.concatenate([a[:, -d:], jnp.zeros_like(a[:, :-d])], axis=1)
    return a


def _start_slab_copy(x_ref, xe_scr, sem, bi, hi, nblk, slot):
    h = x_ref.shape[1]

    @pl.when(hi == 0)
    def _():
        pltpu.make_async_copy(
            x_ref.at[bi, pl.ds(0, SLAB - HALO)],
            xe_scr.at[slot, pl.ds(HALO, SLAB - HALO)],
            sem.at[slot],
        ).start()

    @pl.when(hi == nblk - 1)
    def _():
        pltpu.make_async_copy(
            x_ref.at[bi, pl.ds(h - (SLAB - HALO), SLAB - HALO)],
            xe_scr.at[slot, pl.ds(0, SLAB - HALO)],
            sem.at[slot],
        ).start()

    @pl.when((hi > 0) & (hi < nblk - 1))
    def _():
        pltpu.make_async_copy(
            x_ref.at[bi, pl.ds(hi * HBLK - HALO, SLAB)],
            xe_scr.at[slot, pl.ds(0, SLAB)],
            sem.at[slot],
        ).start()


def _wait_slab_copy(x_ref, xe_scr, sem, hi, nblk, slot):
    @pl.when((hi == 0) | (hi == nblk - 1))
    def _():
        pltpu.make_async_copy(
            x_ref.at[0, pl.ds(0, SLAB - HALO)],
            xe_scr.at[slot, pl.ds(0, SLAB - HALO)],
            sem.at[slot],
        ).wait()

    @pl.when((hi > 0) & (hi < nblk - 1))
    def _():
        pltpu.make_async_copy(
            x_ref.at[0, pl.ds(0, SLAB)],
            xe_scr.at[slot, pl.ds(0, SLAB)],
            sem.at[slot],
        ).wait()


def _fused_kernel(c_ref, anch_ref, wid_ref, x_ref, o_ref, xe_scr, sem):
    bi = pl.program_id(0)
    hi = pl.program_id(1)
    nb = pl.num_programs(0)
    nblk = pl.num_programs(1)
    step = bi * nblk + hi
    slot = jax.lax.rem(step, 2)

    @pl.when(step == 0)
    def _():
        _start_slab_copy(x_ref, xe_scr, sem, bi, hi, nblk, slot)

    nxt = step + 1
    nxt_bi = nxt // nblk
    nxt_hi = jax.lax.rem(nxt, nblk)

    @pl.when(nxt < nb * nblk)
    def _():
        _start_slab_copy(x_ref, xe_scr, sem, nxt_bi, nxt_hi, nblk, 1 - slot)

    _wait_slab_copy(x_ref, xe_scr, sem, hi, nblk, slot)

    xe = jnp.swapaxes(xe_scr[slot], 1, 2)
    mu = jnp.maximum(xe, 0.0) + EPS
    row = jax.lax.broadcasted_iota(jnp.int32, mu.shape, 0)
    oob = ((hi == 0) & (row < HALO)) | ((hi == nblk - 1) & (row >= HALO + HBLK))
    mu = jnp.where(oob, 0.0, mu)

    alpha = jnp.zeros((HBLK,) + mu.shape[1:], mu.dtype)
    k = mu + _shift_w(mu, 1) + _shift_w(mu, -1)
    for r in range(1, MAX_SCALE + 1):
        if r > 1:
            k = k + _shift_w(mu, r) + _shift_w(mu, -r)
        lo = HALO - r
        if r == 1:
            p2 = k[lo:lo + HBLK + 1] + k[lo + 1:lo + HBLK + 2]
            box = p2[:HBLK] + k[lo + 2:lo + 2 + HBLK]
        else:
            psz = k.shape[0] - lo - 1
            p2 = k[lo:lo + psz] + k[lo + 1:lo + 1 + psz]
            p4 = p2[:psz - 2] + p2[2:psz]
            if r == 2:
                box = p4[:HBLK] + k[lo + 4:lo + 4 + HBLK]
            elif r == 3:
                box = p4[:HBLK] + p2[4:4 + HBLK] + k[lo + 6:lo + 6 + HBLK]
            else:
                box = p4[:HBLK] + p4[4:4 + HBLK] + k[lo + 8:lo + 8 + HBLK]
        alpha = alpha + c_ref[r - 1] * jnp.log2(box + EPS)

    acc = jnp.zeros_like(alpha)
    for a in range(NUM_ANCHORS):
        t = (alpha - anch_ref[a]) * wid_ref[a]
        acc = acc + jnp.minimum(jnp.abs(t), 1.0)
    res = (xe[HALO:HALO + HBLK] + float(NUM_ANCHORS)) - acc
    o_ref[0] = jnp.swapaxes(res, 1, 2)


@jax.jit
def kernel(x, scale_w, anchors, widths):
    b, h, w, c = x.shape
    nblk = h // HBLK

    sw = jax.nn.softmax(scale_w)
    log_r = jnp.log(jnp.asarray([2.0 * r + 1.0 for r in range(1, MAX_SCALE + 1)],
                                dtype=x.dtype))
    dev = log_r - jnp.sum(sw * log_r)
    var = jnp.sum(sw * dev * dev)
    coef = (sw * dev) * (float(np.log(2.0)) / (var + EPS))

    anch = jnp.transpose(anchors).reshape(NUM_ANCHORS, 1, c)
    wid = jnp.transpose(widths).reshape(NUM_ANCHORS, 1, c)

    xt = jnp.transpose(x, (0, 1, 3, 2))

    out_t = pl.pallas_call(
        _fused_kernel,
        grid=(b, nblk),
        in_specs=[
            pl.BlockSpec(memory_space=pltpu.SMEM),
            pl.BlockSpec((NUM_ANCHORS, 1, c), lambda bi, hi: (0, 0, 0)),
            pl.BlockSpec((NUM_ANCHORS, 1, c), lambda bi, hi: (0, 0, 0)),
            pl.BlockSpec(memory_space=pl.ANY),
        ],
        out_specs=pl.BlockSpec((1, HBLK, c, w), lambda bi, hi: (bi, hi, 0, 0)),
        out_shape=jax.ShapeDtypeStruct((b, h, c, w), x.dtype),
        scratch_shapes=[
            pltpu.VMEM((2, SLAB, c, w), x.dtype),
            pltpu.SemaphoreType.DMA((2,)),
        ],
    )(coef, anch, wid, xt)
    return jnp.transpose(out_t, (0, 1, 3, 2))
